# initial kernel scaffold (unmeasured)
import jax
import jax.numpy as jnp
from jax import lax
from jax.experimental import pallas as pl
from jax.experimental.pallas import tpu as pltpu

N_DEV = 4
K_CHUNK = 512


def _mlp_body(x_ref, wg_ref, wu_ref, wd_ref, out_ref):
    k = pl.program_id(0)
    gate = jnp.dot(x_ref[...], wg_ref[...], preferred_element_type=jnp.float32)
    up = jnp.dot(x_ref[...], wu_ref[...], preferred_element_type=jnp.float32)
    h = gate * (up * jax.nn.sigmoid(up))
    partial = jnp.dot(h, wd_ref[...], preferred_element_type=jnp.float32)

    @pl.when(k == 0)
    def _():
        out_ref[...] = partial

    @pl.when(k > 0)
    def _():
        out_ref[...] += partial


def _allreduce_body(p_ref, out_ref, comm_ref, send_sems, recv_sems):
    my = lax.axis_index("i")
    right = (my + 1) % N_DEV

    comm_ref[0] = p_ref[...]
    out_ref[...] = p_ref[...]

    for h in range(N_DEV - 1):
        rdma = pltpu.make_async_remote_copy(
            src_ref=comm_ref.at[h],
            dst_ref=comm_ref.at[h + 1],
            send_sem=send_sems.at[h],
            recv_sem=recv_sems.at[h],
            device_id=(right,),
            device_id_type=pl.DeviceIdType.MESH,
        )
        rdma.start()
        rdma.wait()
        out_ref[...] += comm_ref[h + 1]


def kernel(x, Wg, Wu, Wd):
    m, d = x.shape
    h_per = Wg.shape[1]
    n_chunks = h_per // K_CHUNK

    partial = pl.pallas_call(
        _mlp_body,
        grid=(n_chunks,),
        in_specs=[
            pl.BlockSpec((m, d), lambda k: (0, 0)),
            pl.BlockSpec((d, K_CHUNK), lambda k: (0, k)),
            pl.BlockSpec((d, K_CHUNK), lambda k: (0, k)),
            pl.BlockSpec((K_CHUNK, m), lambda k: (k, 0)),
        ],
        out_specs=pl.BlockSpec((m, m), lambda k: (0, 0)),
        out_shape=jax.ShapeDtypeStruct((m, m), jnp.float32),
    )(x, Wg, Wu, Wd)

    return pl.pallas_call(
        _allreduce_body,
        out_shape=jax.ShapeDtypeStruct((m, m), jnp.float32),
        in_specs=[pl.BlockSpec(memory_space=pltpu.VMEM)],
        out_specs=pl.BlockSpec(memory_space=pltpu.VMEM),
        scratch_shapes=[
            pltpu.VMEM((N_DEV, m, m), jnp.float32),
            pltpu.SemaphoreType.DMA((N_DEV - 1,)),
            pltpu.SemaphoreType.DMA((N_DEV - 1,)),
        ],
    )(partial)


# baseline (device time: 398642 ns/iter reference)
import jax
import jax.numpy as jnp
from jax import lax
from jax.experimental import pallas as pl
from jax.experimental.pallas import tpu as pltpu

N_DEV = 4
K_CHUNK = 512


def _mlp_body(x_ref, wg_ref, wu_ref, wd_ref, out_ref):
    k = pl.program_id(0)
    gate = jnp.dot(x_ref[...], wg_ref[...], preferred_element_type=jnp.float32)
    up = jnp.dot(x_ref[...], wu_ref[...], preferred_element_type=jnp.float32)
    h = gate * (up * jax.nn.sigmoid(up))
    partial = jnp.dot(h, wd_ref[...], preferred_element_type=jnp.float32)

    @pl.when(k == 0)
    def _():
        out_ref[...] = partial

    @pl.when(k > 0)
    def _():
        out_ref[...] += partial


def _allreduce_body(p_ref, out_ref, comm_ref, send_sems, recv_sems):
    my = lax.axis_index("i")
    right = (my + 1) % N_DEV

    comm_ref[0] = p_ref[...]
    out_ref[...] = p_ref[...]

    for h in range(N_DEV - 1):
        rdma = pltpu.make_async_remote_copy(
            src_ref=comm_ref.at[h],
            dst_ref=comm_ref.at[h + 1],
            send_sem=send_sems.at[h],
            recv_sem=recv_sems.at[h],
            device_id=(right,),
            device_id_type=pl.DeviceIdType.MESH,
        )
        rdma.start()
        rdma.wait()
        out_ref[...] += comm_ref[h + 1]


def kernel(x, Wg, Wu, Wd):
    m, d = x.shape
    h_per = Wg.shape[1]
    n_chunks = h_per // K_CHUNK

    partial = pl.pallas_call(
        _mlp_body,
        grid=(n_chunks,),
        in_specs=[
            pl.BlockSpec((m, d), lambda k: (0, 0)),
            pl.BlockSpec((d, K_CHUNK), lambda k: (0, k)),
            pl.BlockSpec((d, K_CHUNK), lambda k: (0, k)),
            pl.BlockSpec((K_CHUNK, m), lambda k: (k, 0)),
        ],
        out_specs=pl.BlockSpec((m, m), lambda k: (0, 0)),
        out_shape=jax.ShapeDtypeStruct((m, m), jnp.float32),
        compiler_params=pltpu.CompilerParams(
            vmem_limit_bytes=100 * 1024 * 1024
        ),
    )(x, Wg, Wu, Wd)

    return pl.pallas_call(
        _allreduce_body,
        out_shape=jax.ShapeDtypeStruct((m, m), jnp.float32),
        in_specs=[pl.BlockSpec(memory_space=pltpu.VMEM)],
        out_specs=pl.BlockSpec(memory_space=pltpu.VMEM),
        scratch_shapes=[
            pltpu.VMEM((N_DEV, m, m), jnp.float32),
            pltpu.SemaphoreType.DMA((N_DEV - 1,)),
            pltpu.SemaphoreType.DMA((N_DEV - 1,)),
        ],
        compiler_params=pltpu.CompilerParams(
            vmem_limit_bytes=100 * 1024 * 1024
        ),
    )(partial)


# device time: 249433 ns/iter; 1.5982x vs baseline; 1.5982x over previous
import jax
import jax.numpy as jnp
from jax import lax
from jax.experimental import pallas as pl
from jax.experimental.pallas import tpu as pltpu

N_DEV = 4
K_CHUNK = 512


def _mlp_body(x_ref, wg_ref, wu_ref, wd_ref, out_ref):
    k = pl.program_id(0)
    gate = jnp.dot(x_ref[...], wg_ref[...], preferred_element_type=jnp.float32)
    up = jnp.dot(x_ref[...], wu_ref[...], preferred_element_type=jnp.float32)
    h = gate * (up * jax.nn.sigmoid(up))
    partial = jnp.dot(h, wd_ref[...], preferred_element_type=jnp.float32)

    @pl.when(k == 0)
    def _():
        out_ref[...] = partial

    @pl.when(k > 0)
    def _():
        out_ref[...] += partial


def _allreduce_body(p_ref, out_ref, comm_ref, send_sems, recv_sems):
    blk = p_ref.shape[0] // N_DEV
    my = lax.axis_index("i")
    right = jnp.mod(my + 1, N_DEV)

    for s in range(N_DEV - 1):
        if s == 0:
            send_c = jnp.mod(my - 1, N_DEV)
            src = p_ref.at[pl.ds(send_c * blk, blk), :]
        else:
            src = comm_ref.at[s - 1]
        rdma = pltpu.make_async_remote_copy(
            src_ref=src,
            dst_ref=comm_ref.at[s],
            send_sem=send_sems.at[s],
            recv_sem=recv_sems.at[s],
            device_id=(right,),
            device_id_type=pl.DeviceIdType.MESH,
        )
        rdma.start()
        rdma.wait()
        recv_c = jnp.mod(my - 2 - s, N_DEV)
        comm_ref[s] += p_ref[pl.ds(recv_c * blk, blk), :]

    out_ref[pl.ds(my * blk, blk), :] = comm_ref[N_DEV - 2]

    for t in range(N_DEV - 1):
        rdma = pltpu.make_async_remote_copy(
            src_ref=comm_ref.at[N_DEV - 2 + t],
            dst_ref=comm_ref.at[N_DEV - 1 + t],
            send_sem=send_sems.at[N_DEV - 1 + t],
            recv_sem=recv_sems.at[N_DEV - 1 + t],
            device_id=(right,),
            device_id_type=pl.DeviceIdType.MESH,
        )
        rdma.start()
        rdma.wait()
        recv_c = jnp.mod(my - 1 - t, N_DEV)
        out_ref[pl.ds(recv_c * blk, blk), :] = comm_ref[N_DEV - 1 + t]


def kernel(x, Wg, Wu, Wd):
    m, d = x.shape
    h_per = Wg.shape[1]
    n_chunks = h_per // K_CHUNK

    partial = pl.pallas_call(
        _mlp_body,
        grid=(n_chunks,),
        in_specs=[
            pl.BlockSpec((m, d), lambda k: (0, 0)),
            pl.BlockSpec((d, K_CHUNK), lambda k: (0, k)),
            pl.BlockSpec((d, K_CHUNK), lambda k: (0, k)),
            pl.BlockSpec((K_CHUNK, m), lambda k: (k, 0)),
        ],
        out_specs=pl.BlockSpec((m, m), lambda k: (0, 0)),
        out_shape=jax.ShapeDtypeStruct((m, m), jnp.float32),
        compiler_params=pltpu.CompilerParams(
            vmem_limit_bytes=100 * 1024 * 1024
        ),
    )(x, Wg, Wu, Wd)

    return pl.pallas_call(
        _allreduce_body,
        out_shape=jax.ShapeDtypeStruct((m, m), jnp.float32),
        in_specs=[pl.BlockSpec(memory_space=pltpu.VMEM)],
        out_specs=pl.BlockSpec(memory_space=pltpu.VMEM),
        scratch_shapes=[
            pltpu.VMEM((2 * (N_DEV - 1), m // N_DEV, m), jnp.float32),
            pltpu.SemaphoreType.DMA((2 * (N_DEV - 1),)),
            pltpu.SemaphoreType.DMA((2 * (N_DEV - 1),)),
        ],
        compiler_params=pltpu.CompilerParams(
            vmem_limit_bytes=100 * 1024 * 1024
        ),
    )(partial)


# device time: 165169 ns/iter; 2.4135x vs baseline; 1.5102x over previous
import jax
import jax.numpy as jnp
from jax import lax
from jax.experimental import pallas as pl
from jax.experimental.pallas import tpu as pltpu

N_DEV = 4
K_CHUNK = 512


def _body(
    x_ref, wg_ref, wu_ref, wd_ref, out_ref,
    acc_ref, rs_ref,
    rs_s, rs_r, cw_s, cw_r, ccw_s, ccw_r,
):
    m = x_ref.shape[0]
    blk = m // N_DEV
    hblk = blk // 2
    n_k = pl.num_programs(1)
    c = pl.program_id(0)
    k = pl.program_id(1)
    my = lax.axis_index("i")
    right = jnp.mod(my + 1, N_DEV)
    left = jnp.mod(my - 1, N_DEV)

    @pl.when(jnp.logical_and(c == 0, k == 0))
    def _():
        barrier_sem = pltpu.get_barrier_semaphore()
        for nbr in (left, right):
            pl.semaphore_signal(
                barrier_sem, inc=1,
                device_id=(nbr,), device_id_type=pl.DeviceIdType.MESH,
            )
        pl.semaphore_wait(barrier_sem, 2)

    row_c = jnp.mod(my - 1 - c, N_DEV)
    xa = x_ref[pl.ds(row_c * blk, blk), :]
    gate = jnp.dot(xa, wg_ref[...], preferred_element_type=jnp.float32)
    up = jnp.dot(xa, wu_ref[...], preferred_element_type=jnp.float32)
    h = gate * (up * jax.nn.sigmoid(up))
    pk = jnp.dot(h, wd_ref[...], preferred_element_type=jnp.float32)

    @pl.when(k == 0)
    def _():
        acc_ref[c] = pk

    @pl.when(k > 0)
    def _():
        acc_ref[c] += pk

    def rs_desc(s, src):
        return pltpu.make_async_remote_copy(
            src_ref=src,
            dst_ref=rs_ref.at[s + 1],
            send_sem=rs_s.at[s],
            recv_sem=rs_r.at[s],
            device_id=(right,),
            device_id_type=pl.DeviceIdType.MESH,
        )

    @pl.when(jnp.logical_and(c == 0, k == n_k - 1))
    def _():
        rs_desc(0, acc_ref.at[0]).start()

    for cc in (1, 2):
        @pl.when(jnp.logical_and(c == cc, k == n_k - 1))
        def _(cc=cc):
            rs_desc(cc - 1, acc_ref.at[0]).wait_recv()
            rs_ref[cc] += acc_ref[cc]
            rs_desc(cc, rs_ref.at[cc]).start()

    @pl.when(jnp.logical_and(c == N_DEV - 1, k == n_k - 1))
    def _():
        rs_desc(2, acc_ref.at[0]).wait_recv()
        rs_ref[3] += acc_ref[3]
        out_ref[pl.ds(my * blk, blk), :] = rs_ref[3]

        def ag_desc(t, cw):
            q = jnp.mod(my - t, N_DEV) if cw else jnp.mod(my + t, N_DEV)
            r0 = q * blk + (0 if cw else hblk)
            return pltpu.make_async_remote_copy(
                src_ref=out_ref.at[pl.ds(r0, hblk), :],
                dst_ref=out_ref.at[pl.ds(r0, hblk), :],
                send_sem=(cw_s if cw else ccw_s).at[t],
                recv_sem=(cw_r if cw else ccw_r).at[t],
                device_id=(right if cw else left,),
                device_id_type=pl.DeviceIdType.MESH,
            )

        for t in range(N_DEV - 1):
            if t > 0:
                ag_desc(t - 1, True).wait_recv()
                ag_desc(t - 1, False).wait_recv()
            ag_desc(t, True).start()
            ag_desc(t, False).start()
        ag_desc(N_DEV - 2, True).wait_recv()
        ag_desc(N_DEV - 2, False).wait_recv()

        for s in range(N_DEV - 1):
            rs_desc(s, acc_ref.at[0]).wait_send()
            ag_desc_t = ag_desc(s, True)
            ag_desc_t.wait_send()
            ag_desc(s, False).wait_send()


def kernel(x, Wg, Wu, Wd):
    m, d = x.shape
    h_per = Wg.shape[1]
    n_k = h_per // K_CHUNK
    blk = m // N_DEV

    return pl.pallas_call(
        _body,
        grid=(N_DEV, n_k),
        in_specs=[
            pl.BlockSpec((m, d), lambda c, k: (0, 0)),
            pl.BlockSpec((d, K_CHUNK), lambda c, k: (0, k)),
            pl.BlockSpec((d, K_CHUNK), lambda c, k: (0, k)),
            pl.BlockSpec((K_CHUNK, m), lambda c, k: (k, 0)),
        ],
        out_specs=pl.BlockSpec((m, m), lambda c, k: (0, 0)),
        out_shape=jax.ShapeDtypeStruct((m, m), jnp.float32),
        scratch_shapes=[
            pltpu.VMEM((N_DEV, blk, m), jnp.float32),
            pltpu.VMEM((N_DEV, blk, m), jnp.float32),
            pltpu.SemaphoreType.DMA((N_DEV - 1,)),
            pltpu.SemaphoreType.DMA((N_DEV - 1,)),
            pltpu.SemaphoreType.DMA((N_DEV - 1,)),
            pltpu.SemaphoreType.DMA((N_DEV - 1,)),
            pltpu.SemaphoreType.DMA((N_DEV - 1,)),
            pltpu.SemaphoreType.DMA((N_DEV - 1,)),
        ],
        compiler_params=pltpu.CompilerParams(
            collective_id=0,
            vmem_limit_bytes=100 * 1024 * 1024,
        ),
    )(x, Wg, Wu, Wd)


# device time: 161301 ns/iter; 2.4714x vs baseline; 1.0240x over previous
import jax
import jax.numpy as jnp
from jax import lax
from jax.experimental import pallas as pl
from jax.experimental.pallas import tpu as pltpu

N_DEV = 4
K_CHUNK = 512
MODE = "fused"
ENABLE_COMM = MODE == "fused"


def _body(
    x_ref, wg_ref, wu_ref, wd_ref, out_ref,
    rs_ref,
    rs_s, rs_r, cw_s, cw_r, ccw_s, ccw_r,
):
    m = x_ref.shape[0]
    blk = m // N_DEV
    hblk = blk // 2
    n_k = pl.num_programs(1)
    c = pl.program_id(0)
    k = pl.program_id(1)
    my = lax.axis_index("i")
    right = jnp.mod(my + 1, N_DEV)
    left = jnp.mod(my - 1, N_DEV)

    @pl.when(jnp.logical_and(c == 0, k == 0))
    def _():
        barrier_sem = pltpu.get_barrier_semaphore()
        for nbr in (left, right):
            pl.semaphore_signal(
                barrier_sem, inc=1,
                device_id=(nbr,), device_id_type=pl.DeviceIdType.MESH,
            )
        pl.semaphore_wait(barrier_sem, 2)

    row_c = jnp.mod(my - 1 - c, N_DEV)
    rows = pl.ds(row_c * blk, blk)
    xa = x_ref[rows, :]
    gate = jnp.dot(xa, wg_ref[...], preferred_element_type=jnp.float32)
    up = jnp.dot(xa, wu_ref[...], preferred_element_type=jnp.float32)
    h = gate * (up * jax.nn.sigmoid(up))
    wd_k = wd_ref[pl.ds(k * K_CHUNK, K_CHUNK), :]
    pk = jnp.dot(h, wd_k, preferred_element_type=jnp.float32)

    @pl.when(k == 0)
    def _():
        out_ref[rows, :] = pk

    @pl.when(k > 0)
    def _():
        out_ref[rows, :] += pk

    if not ENABLE_COMM:
        return

    def rs_desc(s, hx, src_ref, src_row):
        return pltpu.make_async_remote_copy(
            src_ref=src_ref.at[pl.ds(src_row + hx * hblk, hblk), :],
            dst_ref=rs_ref.at[s, pl.ds(hx * hblk, hblk), :],
            send_sem=rs_s.at[s, hx],
            recv_sem=rs_r.at[s, hx],
            device_id=(right,),
            device_id_type=pl.DeviceIdType.MESH,
        )

    @pl.when(jnp.logical_and(c == 0, k == n_k - 1))
    def _():
        for hx in (0, 1):
            rs_desc(0, hx, out_ref, row_c * blk).start()

    for cc in (1, 2):
        @pl.when(jnp.logical_and(c == cc, k == n_k - 1))
        def _(cc=cc):
            for hx in (0, 1):
                half = pl.ds(hx * hblk, hblk)
                rs_desc(cc - 1, hx, rs_ref.at[0], 0).wait_recv()
                rs_ref[cc - 1, half, :] += out_ref[
                    pl.ds(row_c * blk + hx * hblk, hblk), :
                ]
                rs_desc(cc, hx, rs_ref.at[cc - 1], 0).start()

    @pl.when(jnp.logical_and(c == N_DEV - 1, k == n_k - 1))
    def _():
        for hx in (0, 1):
            half = pl.ds(hx * hblk, hblk)
            rs_desc(2, hx, rs_ref.at[0], 0).wait_recv()
            rs_ref[2, half, :] += out_ref[
                pl.ds(row_c * blk + hx * hblk, hblk), :
            ]
            out_ref[pl.ds(my * blk + hx * hblk, hblk), :] = rs_ref[2, half, :]

        def ag_desc(stage, cw):
            if stage == 0:
                q, r0, n_rows = my, my * blk, blk
            elif cw:
                q = jnp.mod(my - 1, N_DEV)
                r0, n_rows = q * blk, hblk
            else:
                q = jnp.mod(my + 1, N_DEV)
                r0, n_rows = q * blk + hblk, hblk
            return pltpu.make_async_remote_copy(
                src_ref=out_ref.at[pl.ds(r0, n_rows), :],
                dst_ref=out_ref.at[pl.ds(r0, n_rows), :],
                send_sem=(cw_s if cw else ccw_s).at[stage],
                recv_sem=(cw_r if cw else ccw_r).at[stage],
                device_id=(right if cw else left,),
                device_id_type=pl.DeviceIdType.MESH,
            )

        ag_desc(0, True).start()
        ag_desc(0, False).start()
        ag_desc(0, True).wait_recv()
        ag_desc(0, False).wait_recv()
        ag_desc(1, True).start()
        ag_desc(1, False).start()
        ag_desc(1, True).wait_recv()
        ag_desc(1, False).wait_recv()

        for s in range(N_DEV - 1):
            for hx in (0, 1):
                rs_desc(s, hx, rs_ref.at[0], 0).wait_send()
        for stage in (0, 1):
            ag_desc(stage, True).wait_send()
            ag_desc(stage, False).wait_send()


def _plain_body(x_ref, wg_ref, wu_ref, wd_ref, out_ref):
    k = pl.program_id(0)
    gate = jnp.dot(x_ref[...], wg_ref[...], preferred_element_type=jnp.float32)
    up = jnp.dot(x_ref[...], wu_ref[...], preferred_element_type=jnp.float32)
    h = gate * (up * jax.nn.sigmoid(up))
    pk = jnp.dot(h, wd_ref[...], preferred_element_type=jnp.float32)

    @pl.when(k == 0)
    def _():
        out_ref[...] = pk

    @pl.when(k > 0)
    def _():
        out_ref[...] += pk


def _rowtile_body(x_ref, wg_ref, wu_ref, wd_ref, out_ref, acc_ref):
    k = pl.program_id(0)
    c = pl.program_id(1)
    n_k = pl.num_programs(0)
    blk = acc_ref.shape[1]
    gate = jnp.dot(x_ref[...], wg_ref[...], preferred_element_type=jnp.float32)
    up = jnp.dot(x_ref[...], wu_ref[...], preferred_element_type=jnp.float32)
    h = gate * (up * jax.nn.sigmoid(up))
    pk = jnp.dot(h, wd_ref[...], preferred_element_type=jnp.float32)

    @pl.when(k == 0)
    def _():
        acc_ref[c] = pk

    @pl.when(k > 0)
    def _():
        acc_ref[c] += pk

    @pl.when(k == n_k - 1)
    def _():
        out_ref[pl.ds(c * blk, blk), :] = acc_ref[c]


def kernel(x, Wg, Wu, Wd):
    m, d = x.shape
    h_per = Wg.shape[1]
    n_k = h_per // K_CHUNK
    blk = m // N_DEV

    if MODE == "compute_plain":
        return pl.pallas_call(
            _plain_body,
            grid=(n_k,),
            in_specs=[
                pl.BlockSpec((m, d), lambda k: (0, 0)),
                pl.BlockSpec((d, K_CHUNK), lambda k: (0, k)),
                pl.BlockSpec((d, K_CHUNK), lambda k: (0, k)),
                pl.BlockSpec((K_CHUNK, m), lambda k: (k, 0)),
            ],
            out_specs=pl.BlockSpec((m, m), lambda k: (0, 0)),
            out_shape=jax.ShapeDtypeStruct((m, m), jnp.float32),
            compiler_params=pltpu.CompilerParams(
                vmem_limit_bytes=100 * 1024 * 1024
            ),
        )(x, Wg, Wu, Wd)

    if MODE == "compute_rowtile":
        return pl.pallas_call(
            _rowtile_body,
            grid=(n_k, N_DEV),
            in_specs=[
                pl.BlockSpec((blk, d), lambda k, c: (c, 0)),
                pl.BlockSpec((d, K_CHUNK), lambda k, c: (0, k)),
                pl.BlockSpec((d, K_CHUNK), lambda k, c: (0, k)),
                pl.BlockSpec((K_CHUNK, m), lambda k, c: (k, 0)),
            ],
            out_specs=pl.BlockSpec((m, m), lambda k, c: (0, 0)),
            out_shape=jax.ShapeDtypeStruct((m, m), jnp.float32),
            scratch_shapes=[pltpu.VMEM((N_DEV, blk, m), jnp.float32)],
            compiler_params=pltpu.CompilerParams(
                vmem_limit_bytes=100 * 1024 * 1024
            ),
        )(x, Wg, Wu, Wd)

    return pl.pallas_call(
        _body,
        grid=(N_DEV, n_k),
        in_specs=[
            pl.BlockSpec((m, d), lambda c, k: (0, 0)),
            pl.BlockSpec((d, K_CHUNK), lambda c, k: (0, k)),
            pl.BlockSpec((d, K_CHUNK), lambda c, k: (0, k)),
            pl.BlockSpec((h_per, m), lambda c, k: (0, 0)),
        ],
        out_specs=pl.BlockSpec((m, m), lambda c, k: (0, 0)),
        out_shape=jax.ShapeDtypeStruct((m, m), jnp.float32),
        scratch_shapes=[
            pltpu.VMEM((N_DEV - 1, blk, m), jnp.float32),
            pltpu.SemaphoreType.DMA((N_DEV - 1, 2)),
            pltpu.SemaphoreType.DMA((N_DEV - 1, 2)),
            pltpu.SemaphoreType.DMA((N_DEV - 1,)),
            pltpu.SemaphoreType.DMA((N_DEV - 1,)),
            pltpu.SemaphoreType.DMA((N_DEV - 1,)),
            pltpu.SemaphoreType.DMA((N_DEV - 1,)),
        ],
        compiler_params=pltpu.CompilerParams(
            collective_id=0,
            vmem_limit_bytes=100 * 1024 * 1024,
        ),
    )(x, Wg, Wu, Wd)
